# baseline (speedup 1.0000x reference)
import functools

import jax
import jax.numpy as jnp
from jax import lax
from jax.experimental import pallas as pl
from jax.experimental.pallas import tpu as pltpu
from jax.experimental.pallas import tpu_sc as plsc

H = 1024
E = 64
K = 2
T = 2048
CAP = 40
NSLOT = E * CAP
FF = 2 * H
NC = 1
FC = FF // NC
TC_CH = 256
EPS = 1e-5
CUM_CH = 256

_SC_INFO = plsc.get_sparse_core_info()
SC_NC = _SC_INFO.num_cores
SC_NS = _SC_INFO.num_subcores
NW = SC_NC * SC_NS
NA = K * T
APW = NA // NW
RND = 2
RPR = APW // RND


def _router_kernel(h_ref, rwt_ref, oh_ref, pos_ref, slot_ref, cw_ref,
                   hbf_ref):
    h = h_ref[...]
    logits = jnp.dot(h, rwt_ref[...], preferred_element_type=jnp.float32)

    iota_e = lax.broadcasted_iota(jnp.int32, (T, E), 1)
    m1 = jnp.max(logits, axis=1, keepdims=True)
    a1 = jnp.min(jnp.where(logits == m1, iota_e, E), axis=1, keepdims=True)
    oh1 = iota_e == a1
    logits2 = jnp.where(oh1, -jnp.inf, logits)
    m2 = jnp.max(logits2, axis=1, keepdims=True)
    a2 = jnp.min(jnp.where(logits2 == m2, iota_e, E), axis=1, keepdims=True)
    oh2 = iota_e == a2

    s1 = 1.0 / (1.0 + jnp.exp(m2 - m1))
    s2 = 1.0 - s1

    ohf = (oh1 | oh2).astype(jnp.float32)

    nch = T // CUM_CH
    lt = (lax.broadcasted_iota(jnp.int32, (CUM_CH, CUM_CH), 0)
          > lax.broadcasted_iota(jnp.int32, (CUM_CH, CUM_CH), 1)
          ).astype(jnp.float32)
    run = jnp.zeros((1, E), dtype=jnp.float32)
    chunks = []
    for c in range(nch):
        blk = ohf[c * CUM_CH:(c + 1) * CUM_CH, :]
        chunks.append(jnp.dot(lt, blk, preferred_element_type=jnp.float32)
                      + run)
        run = run + jnp.sum(blk, axis=0, keepdims=True)
    pos = jnp.concatenate(chunks, axis=0)

    p1 = jnp.sum(jnp.where(oh1, pos, 0.0), axis=1, keepdims=True)
    p2 = jnp.sum(jnp.where(oh2, pos, 0.0), axis=1, keepdims=True)
    v1 = p1 < CAP
    v2 = p2 < CAP
    cw1 = jnp.where(v1, s1, 0.0)
    cw2 = jnp.where(v2, s2, 0.0)

    slot1 = jnp.where(v1, a1 * CAP + p1.astype(jnp.int32), 0)
    slot2 = jnp.where(v2, a2 * CAP + p2.astype(jnp.int32), 0)

    iota_k = lax.broadcasted_iota(jnp.int32, (T, K), 1)
    slot_ref[...] = jnp.where(iota_k == 0, slot1, slot2)
    cw_ref[...] = jnp.where(iota_k == 0, cw1, cw2)

    oh_ref[...] = ohf
    pos_ref[...] = pos
    hbf_ref[...] = h.astype(jnp.bfloat16)


def _gelu(x):
    return 0.5 * x * (1.0 + lax.erf(x * 0.7071067811865476))


def _expert_kernel(post_ref, oht_ref, hbf_ref, w1_ref, b1_ref,
                   w2_ref, b2_ref, ybuf_ref, x_scr, y_scr):
    c = pl.program_id(1)

    @pl.when(c == 0)
    def _():
        pos_e = post_ref[0, 0, :].reshape(1, T)
        oh_e = oht_ref[0, 0, :].reshape(1, T)
        iota_p = lax.broadcasted_iota(jnp.int32, (CAP, T), 0).astype(
            jnp.float32)
        d = jnp.where((pos_e == iota_p) & (oh_e > 0.5), 1.0, 0.0)
        x_scr[...] = jnp.dot(d.astype(jnp.bfloat16), hbf_ref[...],
                             preferred_element_type=jnp.float32)

    xw1 = jnp.dot(x_scr[...], w1_ref[0],
                  preferred_element_type=jnp.float32)
    b1c = b1_ref[0, 0, pl.ds(c * FC, FC)].reshape(1, FC)
    h1 = _gelu(xw1 + b1c)
    contrib = jnp.dot(h1, w2_ref[0],
                      preferred_element_type=jnp.float32)

    @pl.when(c == 0)
    def _():
        y_scr[...] = contrib

    @pl.when(c != 0)
    def _():
        y_scr[...] += contrib

    @pl.when(c == NC - 1)
    def _():
        ybuf_ref[...] = y_scr[...] + b2_ref[0, 0, :].reshape(1, H)


def _sc_combine(sflat_hbm, ybuf_hbm, r_hbm, idx_v, rows_v, sem):
    wid = lax.axis_index("s") * SC_NC + lax.axis_index("c")
    base = wid * APW
    for rnd in range(RND):
        pltpu.sync_copy(sflat_hbm.at[pl.ds(base + rnd * RPR, RPR)], idx_v)
        pltpu.async_copy(ybuf_hbm.at[idx_v], rows_v, sem).wait()
        pltpu.sync_copy(rows_v, r_hbm.at[pl.ds(base + rnd * RPR, RPR)])


def _ln_kernel(h_ref, r_ref, cw_ref, g_ref, beta_ref, out_ref):
    r0 = r_ref[0]
    r1 = r_ref[1]
    cw = cw_ref[...]
    moe = cw[:, 0:1] * r0 + cw[:, 1:2] * r1
    resid = h_ref[...] + moe
    mean = jnp.mean(resid, axis=1, keepdims=True)
    cent = resid - mean
    var = jnp.mean(cent * cent, axis=1, keepdims=True)
    normed = cent / jnp.sqrt(var + EPS)
    out_ref[...] = normed * g_ref[0, :].reshape(1, H) \
        + beta_ref[0, :].reshape(1, H)


@jax.jit
def _moe_pallas(h2d, rwt, w1, b1r, w2, b2r, g2, beta2):
    oh, pos, slots, cw, hbf = pl.pallas_call(
        _router_kernel,
        out_shape=[
            jax.ShapeDtypeStruct((T, E), jnp.float32),
            jax.ShapeDtypeStruct((T, E), jnp.float32),
            jax.ShapeDtypeStruct((T, K), jnp.int32),
            jax.ShapeDtypeStruct((T, K), jnp.float32),
            jax.ShapeDtypeStruct((T, H), jnp.bfloat16),
        ],
    )(h2d, rwt)

    post = pos.T.reshape(E, 1, T)
    oht = oh.T.reshape(E, 1, T)
    sflat = slots.T.reshape(NA)

    ybuf = pl.pallas_call(
        _expert_kernel,
        grid=(E, NC),
        in_specs=[
            pl.BlockSpec((1, 1, T), lambda e, c: (e, 0, 0)),
            pl.BlockSpec((1, 1, T), lambda e, c: (e, 0, 0)),
            pl.BlockSpec((T, H), lambda e, c: (0, 0)),
            pl.BlockSpec((1, H, FC), lambda e, c: (e, 0, c)),
            pl.BlockSpec((1, 1, FF), lambda e, c: (e, 0, 0)),
            pl.BlockSpec((1, FC, H), lambda e, c: (e, c, 0)),
            pl.BlockSpec((1, 1, H), lambda e, c: (e, 0, 0)),
        ],
        out_specs=pl.BlockSpec((CAP, H), lambda e, c: (e, 0)),
        out_shape=jax.ShapeDtypeStruct((NSLOT, H), jnp.float32),
        scratch_shapes=[
            pltpu.VMEM((CAP, H), jnp.float32),
            pltpu.VMEM((CAP, H), jnp.float32),
        ],
        compiler_params=pltpu.CompilerParams(
            dimension_semantics=("arbitrary", "arbitrary"),
        ),
    )(post, oht, hbf, w1, b1r, w2, b2r)

    sc_combine = functools.partial(
        pl.kernel,
        mesh=plsc.VectorSubcoreMesh(core_axis_name="c", subcore_axis_name="s"),
        out_type=jax.ShapeDtypeStruct((NA, H), jnp.float32),
        scratch_types=[
            pltpu.VMEM((RPR,), jnp.int32),
            pltpu.VMEM((RPR, H), jnp.float32),
            pltpu.SemaphoreType.DMA,
        ],
    )(_sc_combine)
    r = sc_combine(sflat, ybuf).reshape(K, T, H)

    out = pl.pallas_call(
        _ln_kernel,
        grid=(T // TC_CH,),
        in_specs=[
            pl.BlockSpec((TC_CH, H), lambda t: (t, 0)),
            pl.BlockSpec((K, TC_CH, H), lambda t: (0, t, 0)),
            pl.BlockSpec((TC_CH, K), lambda t: (t, 0)),
            pl.BlockSpec((1, H), lambda t: (0, 0)),
            pl.BlockSpec((1, H), lambda t: (0, 0)),
        ],
        out_specs=pl.BlockSpec((TC_CH, H), lambda t: (t, 0)),
        out_shape=jax.ShapeDtypeStruct((T, H), jnp.float32),
        compiler_params=pltpu.CompilerParams(
            dimension_semantics=("arbitrary",),
        ),
    )(h2d, r, cw, g2, beta2)
    return out


def kernel(hidden_states, router_w, w1, b1, w2, b2, ln_gamma, ln_beta):
    B, S, _ = hidden_states.shape
    h2d = hidden_states.reshape(T, H)
    rwt = router_w.T
    b1r = b1.reshape(E, 1, FF)
    b2r = b2.reshape(E, 1, H)
    g2 = ln_gamma.reshape(1, H)
    beta2 = ln_beta.reshape(1, H)
    out = _moe_pallas(h2d, rwt, w1, b1r, w2, b2r, g2, beta2)
    return out.reshape(B, S, H)


# speedup vs baseline: 1.2897x; 1.2897x over previous
import jax
import jax.numpy as jnp
from jax import lax
from jax.experimental import pallas as pl
from jax.experimental.pallas import tpu as pltpu

H = 1024
E = 64
K = 2
T = 2048
CAP = 40
NSLOT = E * CAP
FF = 2 * H
TC_CH = 256
NSTEP = E + T // TC_CH
EPS = 1e-5
CUM_CH = 256


def _router_kernel(h_ref, rwt_ref, oh_ref, pos_ref, wtok_ref, hbf_ref):
    h = h_ref[...]
    logits = jnp.dot(h, rwt_ref[...], preferred_element_type=jnp.float32)

    iota_e = lax.broadcasted_iota(jnp.int32, (T, E), 1)
    m1 = jnp.max(logits, axis=1, keepdims=True)
    a1 = jnp.min(jnp.where(logits == m1, iota_e, E), axis=1, keepdims=True)
    oh1 = iota_e == a1
    logits2 = jnp.where(oh1, -jnp.inf, logits)
    m2 = jnp.max(logits2, axis=1, keepdims=True)
    a2 = jnp.min(jnp.where(logits2 == m2, iota_e, E), axis=1, keepdims=True)
    oh2 = iota_e == a2

    s1 = 1.0 / (1.0 + jnp.exp(m2 - m1))
    s2 = 1.0 - s1

    ohf = (oh1 | oh2).astype(jnp.float32)

    nch = T // CUM_CH
    lt = (lax.broadcasted_iota(jnp.int32, (CUM_CH, CUM_CH), 0)
          > lax.broadcasted_iota(jnp.int32, (CUM_CH, CUM_CH), 1)
          ).astype(jnp.float32)
    run = jnp.zeros((1, E), dtype=jnp.float32)
    chunks = []
    for c in range(nch):
        blk = ohf[c * CUM_CH:(c + 1) * CUM_CH, :]
        chunks.append(jnp.dot(lt, blk, preferred_element_type=jnp.float32)
                      + run)
        run = run + jnp.sum(blk, axis=0, keepdims=True)
    pos = jnp.concatenate(chunks, axis=0)

    p1 = jnp.sum(jnp.where(oh1, pos, 0.0), axis=1, keepdims=True)
    p2 = jnp.sum(jnp.where(oh2, pos, 0.0), axis=1, keepdims=True)
    cw1 = jnp.where(p1 < CAP, s1, 0.0)
    cw2 = jnp.where(p2 < CAP, s2, 0.0)
    wtok = oh1.astype(jnp.float32) * cw1 + oh2.astype(jnp.float32) * cw2

    oh_ref[...] = ohf
    pos_ref[...] = pos
    wtok_ref[...] = wtok
    hbf_ref[...] = h.astype(jnp.bfloat16)


def _gelu(x):
    return 0.5 * x * (1.0 + lax.erf(x * 0.7071067811865476))


def _expert_kernel(post_ref, oht_ref, wtokt_ref, hbf_ref, hblk_ref, w1_ref,
                   b1_ref, w2_ref, b2_ref, g_ref, beta_ref, out_ref,
                   ybuf_scr, dall_scr):
    e = pl.program_id(0)

    @pl.when(e < E)
    def _():
        pos_e = post_ref[0, 0, :].reshape(1, T)
        oh_e = oht_ref[0, 0, :].reshape(1, T)
        wt_e = wtokt_ref[0, 0, :].reshape(1, T)
        iota_p = lax.broadcasted_iota(jnp.int32, (CAP, T), 0).astype(
            jnp.float32)
        d = jnp.where((pos_e == iota_p) & (oh_e > 0.5), 1.0, 0.0)
        dall_scr[pl.ds(e * CAP, CAP), :] = (d * wt_e).astype(jnp.bfloat16)
        x = jnp.dot(d.astype(jnp.bfloat16), hbf_ref[...],
                    preferred_element_type=jnp.float32)
        h1 = _gelu(jnp.dot(x, w1_ref[0], preferred_element_type=jnp.float32)
                   + b1_ref[0, 0, :].reshape(1, FF))
        y = jnp.dot(h1, w2_ref[0], preferred_element_type=jnp.float32) \
            + b2_ref[0, 0, :].reshape(1, H)
        ybuf_scr[pl.ds(e * CAP, CAP), :] = y.astype(jnp.bfloat16)

    @pl.when(e >= E)
    def _():
        tc = e - E
        dchunk = dall_scr[:, pl.ds(tc * TC_CH, TC_CH)]
        moe = lax.dot_general(
            dchunk, ybuf_scr[...], (((0,), (0,)), ((), ())),
            preferred_element_type=jnp.float32)
        resid = hblk_ref[...] + moe
        mean = jnp.mean(resid, axis=1, keepdims=True)
        cent = resid - mean
        var = jnp.mean(cent * cent, axis=1, keepdims=True)
        normed = cent / jnp.sqrt(var + EPS)
        out_ref[...] = normed * g_ref[0, :].reshape(1, H) \
            + beta_ref[0, :].reshape(1, H)


@jax.jit
def _moe_pallas(h2d, rwt, w1, b1r, w2, b2r, g2, beta2):
    oh, pos, wtok, hbf = pl.pallas_call(
        _router_kernel,
        out_shape=[
            jax.ShapeDtypeStruct((T, E), jnp.float32),
            jax.ShapeDtypeStruct((T, E), jnp.float32),
            jax.ShapeDtypeStruct((T, E), jnp.float32),
            jax.ShapeDtypeStruct((T, H), jnp.bfloat16),
        ],
    )(h2d, rwt)

    post = pos.T.reshape(E, 1, T)
    oht = oh.T.reshape(E, 1, T)
    wtokt = wtok.T.reshape(E, 1, T)

    clamp = lambda e: jnp.minimum(e, E - 1)

    out = pl.pallas_call(
        _expert_kernel,
        grid=(NSTEP,),
        in_specs=[
            pl.BlockSpec((1, 1, T), lambda e: (clamp(e), 0, 0)),
            pl.BlockSpec((1, 1, T), lambda e: (clamp(e), 0, 0)),
            pl.BlockSpec((1, 1, T), lambda e: (clamp(e), 0, 0)),
            pl.BlockSpec((T, H), lambda e: (0, 0)),
            pl.BlockSpec((TC_CH, H),
                         lambda e: (jnp.maximum(e - E, 0), 0)),
            pl.BlockSpec((1, H, FF), lambda e: (clamp(e), 0, 0)),
            pl.BlockSpec((1, 1, FF), lambda e: (clamp(e), 0, 0)),
            pl.BlockSpec((1, FF, H), lambda e: (clamp(e), 0, 0)),
            pl.BlockSpec((1, 1, H), lambda e: (clamp(e), 0, 0)),
            pl.BlockSpec((1, H), lambda e: (0, 0)),
            pl.BlockSpec((1, H), lambda e: (0, 0)),
        ],
        out_specs=pl.BlockSpec((TC_CH, H), lambda e: (jnp.maximum(e - E, 0),
                                                      0)),
        out_shape=jax.ShapeDtypeStruct((T, H), jnp.float32),
        scratch_shapes=[
            pltpu.VMEM((NSLOT, H), jnp.bfloat16),
            pltpu.VMEM((NSLOT, T), jnp.bfloat16),
        ],
        compiler_params=pltpu.CompilerParams(
            dimension_semantics=("arbitrary",),
        ),
    )(post, oht, wtokt, hbf, h2d, w1, b1r, w2, b2r, g2, beta2)
    return out


def kernel(hidden_states, router_w, w1, b1, w2, b2, ln_gamma, ln_beta):
    B, S, _ = hidden_states.shape
    h2d = hidden_states.reshape(T, H)
    rwt = router_w.T
    b1r = b1.reshape(E, 1, FF)
    b2r = b2.reshape(E, 1, H)
    g2 = ln_gamma.reshape(1, H)
    beta2 = ln_beta.reshape(1, H)
    out = _moe_pallas(h2d, rwt, w1, b1r, w2, b2r, g2, beta2)
    return out.reshape(B, S, H)
